# initial kernel scaffold (unmeasured)
import jax
import jax.numpy as jnp
from jax import lax
from jax.experimental import pallas as pl
from jax.experimental.pallas import tpu as pltpu

B, SQ, H, D = 4, 32, 8, 128
BH = B // 2
SKV = 4096
SCALE = D ** -0.5


def kernel(Q, K, V):
    def body(q_ref, k_hbm, v_hbm, out_ref,
             k_buf, v_buf, o_part, o_xrecv, o_comb, o_yrecv,
             st_send, st_recv,
             kv_sems,
             ox_send, ox_recv, sx_send, sx_recv, oy_send, oy_recv):
        my_x = lax.axis_index("x")
        my_y = lax.axis_index("y")
        b0 = my_y * BH

        barrier = pltpu.get_barrier_semaphore()
        for nbr in ((1 - my_x, my_y), (my_x, 1 - my_y)):
            pl.semaphore_signal(barrier, inc=1, device_id=nbr,
                                device_id_type=pl.DeviceIdType.MESH)
        pl.semaphore_wait(barrier, 2)

        q_half = q_ref[pl.ds(b0, BH)]

        for b in range(BH):
            for h in range(H):
                ck = pltpu.make_async_copy(
                    k_hbm.at[b0 + b, :, h, :], k_buf, kv_sems.at[0])
                cv = pltpu.make_async_copy(
                    v_hbm.at[b0 + b, :, h, :], v_buf, kv_sems.at[1])
                ck.start()
                cv.start()
                ck.wait()
                cv.wait()
                q = q_half[b, :, h, :]
                s = lax.dot_general(
                    q, k_buf[...], (((1,), (1,)), ((), ())),
                    preferred_element_type=jnp.float32) * SCALE
                m = jnp.max(s, axis=1, keepdims=True)
                p = jnp.exp(s - m)
                l = jnp.sum(p, axis=1, keepdims=True)
                o = lax.dot_general(
                    p, v_buf[...], (((1,), (0,)), ((), ())),
                    preferred_element_type=jnp.float32)
                o_part[b, :, h, :] = o
                st_send[b, :, pl.ds(h, 1)] = m
                st_send[b, :, pl.ds(H + h, 1)] = l

        ox = pltpu.make_async_remote_copy(
            src_ref=o_part, dst_ref=o_xrecv,
            send_sem=ox_send, recv_sem=ox_recv,
            device_id=(1 - my_x, my_y), device_id_type=pl.DeviceIdType.MESH)
        sx = pltpu.make_async_remote_copy(
            src_ref=st_send, dst_ref=st_recv,
            send_sem=sx_send, recv_sem=sx_recv,
            device_id=(1 - my_x, my_y), device_id_type=pl.DeviceIdType.MESH)
        ox.start()
        sx.start()
        ox.wait()
        sx.wait()

        for b in range(BH):
            for h in range(H):
                m1 = st_send[b, :, pl.ds(h, 1)]
                l1 = st_send[b, :, pl.ds(H + h, 1)]
                m2 = st_recv[b, :, pl.ds(h, 1)]
                l2 = st_recv[b, :, pl.ds(H + h, 1)]
                mn = jnp.maximum(m1, m2)
                a1 = jnp.exp(m1 - mn)
                a2 = jnp.exp(m2 - mn)
                ln = a1 * l1 + a2 * l2
                o_comb[b, :, h, :] = (
                    a1 * o_part[b, :, h, :] + a2 * o_xrecv[b, :, h, :]) / ln

        out_ref[pl.ds(b0, BH)] = o_comb[...]

        oy = pltpu.make_async_remote_copy(
            src_ref=o_comb, dst_ref=o_yrecv,
            send_sem=oy_send, recv_sem=oy_recv,
            device_id=(my_x, 1 - my_y), device_id_type=pl.DeviceIdType.MESH)
        oy.start()
        oy.wait()
        out_ref[pl.ds(B - BH - b0, BH)] = o_yrecv[...]

    return pl.pallas_call(
        body,
        out_shape=jax.ShapeDtypeStruct((B, SQ, H, D), jnp.float32),
        in_specs=[
            pl.BlockSpec(memory_space=pltpu.VMEM),
            pl.BlockSpec(memory_space=pltpu.ANY),
            pl.BlockSpec(memory_space=pltpu.ANY),
        ],
        out_specs=pl.BlockSpec(memory_space=pltpu.VMEM),
        scratch_shapes=[
            pltpu.VMEM((SKV, D), jnp.float32),
            pltpu.VMEM((SKV, D), jnp.float32),
            pltpu.VMEM((BH, SQ, H, D), jnp.float32),
            pltpu.VMEM((BH, SQ, H, D), jnp.float32),
            pltpu.VMEM((BH, SQ, H, D), jnp.float32),
            pltpu.VMEM((BH, SQ, H, D), jnp.float32),
            pltpu.VMEM((BH, SQ, 2 * H), jnp.float32),
            pltpu.VMEM((BH, SQ, 2 * H), jnp.float32),
            pltpu.SemaphoreType.DMA((2,)),
            pltpu.SemaphoreType.DMA,
            pltpu.SemaphoreType.DMA,
            pltpu.SemaphoreType.DMA,
            pltpu.SemaphoreType.DMA,
            pltpu.SemaphoreType.DMA,
            pltpu.SemaphoreType.DMA,
        ],
        compiler_params=pltpu.CompilerParams(collective_id=0),
    )(Q, K, V)


# baseline (device time: 63444 ns/iter reference)
import jax
import jax.numpy as jnp
from jax import lax
from jax.experimental import pallas as pl
from jax.experimental.pallas import tpu as pltpu

B, SQ, H, D = 4, 32, 8, 128
BH = B // 2
SKV = 4096
SCALE = D ** -0.5


def kernel(Q, K, V):
    def body(q_ref, k_hbm, v_hbm, out_ref,
             k_buf, v_buf, o_part, o_xrecv, o_comb, o_yrecv,
             st_send, st_recv,
             kv_sems,
             ox_send, ox_recv, sx_send, sx_recv, oy_send, oy_recv):
        my_x = lax.axis_index("x")
        my_y = lax.axis_index("y")
        b0 = my_y * BH

        barrier = pltpu.get_barrier_semaphore()
        for nbr in ((1 - my_x, my_y), (my_x, 1 - my_y)):
            pl.semaphore_signal(barrier, inc=1, device_id=nbr,
                                device_id_type=pl.DeviceIdType.MESH)
        pl.semaphore_wait(barrier, 2)

        q_half = q_ref[pl.ds(b0, BH)]

        for b in range(BH):
            for h in range(H):
                ck = pltpu.make_async_copy(
                    k_hbm.at[b0 + b, :, h, :], k_buf, kv_sems.at[0])
                cv = pltpu.make_async_copy(
                    v_hbm.at[b0 + b, :, h, :], v_buf, kv_sems.at[1])
                ck.start()
                cv.start()
                ck.wait()
                cv.wait()
                q = q_half[b, :, h, :]
                s = lax.dot_general(
                    q, k_buf[...], (((1,), (1,)), ((), ())),
                    preferred_element_type=jnp.float32) * SCALE
                m = jnp.max(s, axis=1, keepdims=True)
                p = jnp.exp(s - m)
                l = jnp.sum(p, axis=1, keepdims=True)
                o = lax.dot_general(
                    p, v_buf[...], (((1,), (0,)), ((), ())),
                    preferred_element_type=jnp.float32)
                o_part[b, :, h, :] = o
                st_send[b, :, pl.ds(h, 1)] = m
                st_send[b, :, pl.ds(H + h, 1)] = l

        ox = pltpu.make_async_remote_copy(
            src_ref=o_part, dst_ref=o_xrecv,
            send_sem=ox_send, recv_sem=ox_recv,
            device_id=(1 - my_x, my_y), device_id_type=pl.DeviceIdType.MESH)
        sx = pltpu.make_async_remote_copy(
            src_ref=st_send, dst_ref=st_recv,
            send_sem=sx_send, recv_sem=sx_recv,
            device_id=(1 - my_x, my_y), device_id_type=pl.DeviceIdType.MESH)
        ox.start()
        sx.start()
        ox.wait()
        sx.wait()

        for b in range(BH):
            for h in range(H):
                m1 = st_send[b, :, pl.ds(h, 1)]
                l1 = st_send[b, :, pl.ds(H + h, 1)]
                m2 = st_recv[b, :, pl.ds(h, 1)]
                l2 = st_recv[b, :, pl.ds(H + h, 1)]
                mn = jnp.maximum(m1, m2)
                a1 = jnp.exp(m1 - mn)
                a2 = jnp.exp(m2 - mn)
                ln = a1 * l1 + a2 * l2
                o_comb[b, :, h, :] = (
                    a1 * o_part[b, :, h, :] + a2 * o_xrecv[b, :, h, :]) / ln

        out_ref[pl.ds(b0, BH)] = o_comb[...]

        oy = pltpu.make_async_remote_copy(
            src_ref=o_comb, dst_ref=o_yrecv,
            send_sem=oy_send, recv_sem=oy_recv,
            device_id=(my_x, 1 - my_y), device_id_type=pl.DeviceIdType.MESH)
        oy.start()
        oy.wait()
        out_ref[pl.ds(B - BH - b0, BH)] = o_yrecv[...]

    return pl.pallas_call(
        body,
        out_shape=jax.ShapeDtypeStruct((B, SQ, H, D), jnp.float32),
        in_specs=[
            pl.BlockSpec(memory_space=pltpu.VMEM),
            pl.BlockSpec(memory_space=pl.ANY),
            pl.BlockSpec(memory_space=pl.ANY),
        ],
        out_specs=pl.BlockSpec(memory_space=pltpu.VMEM),
        scratch_shapes=[
            pltpu.VMEM((SKV, D), jnp.float32),
            pltpu.VMEM((SKV, D), jnp.float32),
            pltpu.VMEM((BH, SQ, H, D), jnp.float32),
            pltpu.VMEM((BH, SQ, H, D), jnp.float32),
            pltpu.VMEM((BH, SQ, H, D), jnp.float32),
            pltpu.VMEM((BH, SQ, H, D), jnp.float32),
            pltpu.VMEM((BH, SQ, 2 * H), jnp.float32),
            pltpu.VMEM((BH, SQ, 2 * H), jnp.float32),
            pltpu.SemaphoreType.DMA((2,)),
            pltpu.SemaphoreType.DMA,
            pltpu.SemaphoreType.DMA,
            pltpu.SemaphoreType.DMA,
            pltpu.SemaphoreType.DMA,
            pltpu.SemaphoreType.DMA,
            pltpu.SemaphoreType.DMA,
        ],
        compiler_params=pltpu.CompilerParams(collective_id=0),
    )(Q, K, V)


# device time: 40855 ns/iter; 1.5529x vs baseline; 1.5529x over previous
import jax
import jax.numpy as jnp
from jax import lax
from jax.experimental import pallas as pl
from jax.experimental.pallas import tpu as pltpu

B, SQ, H, D = 4, 32, 8, 128
BH = B // 2
SKV = 4096
SCALE = D ** -0.5


def kernel(Q, K, V):
    def body(q_ref, k_hbm, v_hbm, out_ref,
             k_buf, v_buf, o_part, o_xrecv, o_comb, o_yrecv,
             st_send, st_recv,
             kv_sems,
             ox_send, ox_recv, sx_send, sx_recv, oy_send, oy_recv):
        my_x = lax.axis_index("x")
        my_y = lax.axis_index("y")
        b0 = my_y * BH

        barrier = pltpu.get_barrier_semaphore()
        for nbr in ((1 - my_x, my_y), (my_x, 1 - my_y)):
            pl.semaphore_signal(barrier, inc=1, device_id=nbr,
                                device_id_type=pl.DeviceIdType.MESH)

        q_half = q_ref[pl.ds(b0, BH)]

        n_it = BH * H

        def start_kv(it):
            b, h = divmod(it, H)
            slot = it % 2
            ck = pltpu.make_async_copy(
                k_hbm.at[b0 + b, :, h, :], k_buf.at[slot],
                kv_sems.at[slot, 0])
            cv = pltpu.make_async_copy(
                v_hbm.at[b0 + b, :, h, :], v_buf.at[slot],
                kv_sems.at[slot, 1])
            ck.start()
            cv.start()
            return ck, cv

        inflight = {0: start_kv(0)}
        for it in range(n_it):
            b, h = divmod(it, H)
            slot = it % 2
            if it + 1 < n_it:
                inflight[it + 1] = start_kv(it + 1)
            ck, cv = inflight.pop(it)
            ck.wait()
            cv.wait()
            q = q_half[b, :, h, :]
            s = lax.dot_general(
                q, k_buf[slot], (((1,), (1,)), ((), ())),
                preferred_element_type=jnp.float32) * SCALE
            m = jnp.max(s, axis=1, keepdims=True)
            p = jnp.exp(s - m)
            l = jnp.sum(p, axis=1, keepdims=True)
            o = lax.dot_general(
                p, v_buf[slot], (((1,), (0,)), ((), ())),
                preferred_element_type=jnp.float32)
            o_part[b, :, h, :] = o
            st_send[b, :, pl.ds(h, 1)] = m
            st_send[b, :, pl.ds(H + h, 1)] = l

        pl.semaphore_wait(barrier, 2)
        ox = pltpu.make_async_remote_copy(
            src_ref=o_part, dst_ref=o_xrecv,
            send_sem=ox_send, recv_sem=ox_recv,
            device_id=(1 - my_x, my_y), device_id_type=pl.DeviceIdType.MESH)
        sx = pltpu.make_async_remote_copy(
            src_ref=st_send, dst_ref=st_recv,
            send_sem=sx_send, recv_sem=sx_recv,
            device_id=(1 - my_x, my_y), device_id_type=pl.DeviceIdType.MESH)
        ox.start()
        sx.start()
        ox.wait()
        sx.wait()

        for b in range(BH):
            for h in range(H):
                m1 = st_send[b, :, pl.ds(h, 1)]
                l1 = st_send[b, :, pl.ds(H + h, 1)]
                m2 = st_recv[b, :, pl.ds(h, 1)]
                l2 = st_recv[b, :, pl.ds(H + h, 1)]
                mn = jnp.maximum(m1, m2)
                a1 = jnp.exp(m1 - mn)
                a2 = jnp.exp(m2 - mn)
                ln = a1 * l1 + a2 * l2
                o_comb[b, :, h, :] = (
                    a1 * o_part[b, :, h, :] + a2 * o_xrecv[b, :, h, :]) / ln

        out_ref[pl.ds(b0, BH)] = o_comb[...]

        oy = pltpu.make_async_remote_copy(
            src_ref=o_comb, dst_ref=o_yrecv,
            send_sem=oy_send, recv_sem=oy_recv,
            device_id=(my_x, 1 - my_y), device_id_type=pl.DeviceIdType.MESH)
        oy.start()
        oy.wait()
        out_ref[pl.ds(B - BH - b0, BH)] = o_yrecv[...]

    return pl.pallas_call(
        body,
        out_shape=jax.ShapeDtypeStruct((B, SQ, H, D), jnp.float32),
        in_specs=[
            pl.BlockSpec(memory_space=pltpu.VMEM),
            pl.BlockSpec(memory_space=pl.ANY),
            pl.BlockSpec(memory_space=pl.ANY),
        ],
        out_specs=pl.BlockSpec(memory_space=pltpu.VMEM),
        scratch_shapes=[
            pltpu.VMEM((2, SKV, D), jnp.float32),
            pltpu.VMEM((2, SKV, D), jnp.float32),
            pltpu.VMEM((BH, SQ, H, D), jnp.float32),
            pltpu.VMEM((BH, SQ, H, D), jnp.float32),
            pltpu.VMEM((BH, SQ, H, D), jnp.float32),
            pltpu.VMEM((BH, SQ, H, D), jnp.float32),
            pltpu.VMEM((BH, SQ, 2 * H), jnp.float32),
            pltpu.VMEM((BH, SQ, 2 * H), jnp.float32),
            pltpu.SemaphoreType.DMA((2, 2)),
            pltpu.SemaphoreType.DMA,
            pltpu.SemaphoreType.DMA,
            pltpu.SemaphoreType.DMA,
            pltpu.SemaphoreType.DMA,
            pltpu.SemaphoreType.DMA,
            pltpu.SemaphoreType.DMA,
        ],
        compiler_params=pltpu.CompilerParams(collective_id=0),
    )(Q, K, V)


# device time: 34538 ns/iter; 1.8369x vs baseline; 1.1829x over previous
import jax
import jax.numpy as jnp
from jax import lax
from jax.experimental import pallas as pl
from jax.experimental.pallas import tpu as pltpu

B, SQ, H, D = 4, 32, 8, 128
BH = B // 2
SKV = 4096
SCALE = D ** -0.5


def kernel(Q, K, V):
    def body(q_ref, k_hbm, v_hbm, out_ref,
             k_buf, v_buf, o_part, o_xrecv, o_comb, o_yrecv,
             st_send, st_recv,
             kv_sems, ox_sems, sx_sems, oy_sems):
        my_x = lax.axis_index("x")
        my_y = lax.axis_index("y")
        b0 = my_y * BH

        barrier = pltpu.get_barrier_semaphore()
        for nbr in ((1 - my_x, my_y), (my_x, 1 - my_y)):
            pl.semaphore_signal(barrier, inc=1, device_id=nbr,
                                device_id_type=pl.DeviceIdType.MESH)

        q_half = q_ref[pl.ds(b0, BH)]

        n_it = BH * H

        HALF = SKV // 2

        def start_kv(it):
            b, h = divmod(it, H)
            slot = it % 4
            cps = []
            for src, dst, j in ((k_hbm, k_buf, 0), (v_hbm, v_buf, 1)):
                for half in range(2):
                    cp = pltpu.make_async_copy(
                        src.at[b0 + b, pl.ds(half * HALF, HALF), h, :],
                        dst.at[slot, pl.ds(half * HALF, HALF)],
                        kv_sems.at[slot, j, half])
                    cp.start()
                    cps.append(cp)
            return cps

        def x_exchange(b):
            ox = pltpu.make_async_remote_copy(
                src_ref=o_part.at[pl.ds(b, 1)],
                dst_ref=o_xrecv.at[pl.ds(b, 1)],
                send_sem=ox_sems.at[b, 0], recv_sem=ox_sems.at[b, 1],
                device_id=(1 - my_x, my_y),
                device_id_type=pl.DeviceIdType.MESH)
            sx = pltpu.make_async_remote_copy(
                src_ref=st_send.at[pl.ds(b, 1)],
                dst_ref=st_recv.at[pl.ds(b, 1)],
                send_sem=sx_sems.at[b, 0], recv_sem=sx_sems.at[b, 1],
                device_id=(1 - my_x, my_y),
                device_id_type=pl.DeviceIdType.MESH)
            ox.start()
            sx.start()
            return ox, sx

        pl.semaphore_wait(barrier, 2)

        x_rdmas = []
        inflight = {i: start_kv(i) for i in range(3)}
        for it in range(n_it):
            b, h = divmod(it, H)
            slot = it % 4
            if it + 3 < n_it:
                inflight[it + 3] = start_kv(it + 3)
            for cp in inflight.pop(it):
                cp.wait()
            q = q_half[b, :, h, :]
            s = lax.dot_general(
                q, k_buf[slot], (((1,), (1,)), ((), ())),
                preferred_element_type=jnp.float32) * SCALE
            m = jnp.max(s, axis=1, keepdims=True)
            p = jnp.exp(s - m)
            l = jnp.sum(p, axis=1, keepdims=True)
            o = lax.dot_general(
                p, v_buf[slot], (((1,), (0,)), ((), ())),
                preferred_element_type=jnp.float32)
            o_part[b, :, h, :] = o
            st_send[b, :, pl.ds(h, 1)] = m
            st_send[b, :, pl.ds(H + h, 1)] = l
            if h == H - 1:
                x_rdmas.append(x_exchange(b))

        y_rdmas = []
        for b in range(BH):
            ox, sx = x_rdmas[b]
            ox.wait_recv()
            sx.wait_recv()
            for h in range(H):
                m1 = st_send[b, :, pl.ds(h, 1)]
                l1 = st_send[b, :, pl.ds(H + h, 1)]
                m2 = st_recv[b, :, pl.ds(h, 1)]
                l2 = st_recv[b, :, pl.ds(H + h, 1)]
                mn = jnp.maximum(m1, m2)
                a1 = jnp.exp(m1 - mn)
                a2 = jnp.exp(m2 - mn)
                ln = a1 * l1 + a2 * l2
                o_comb[b, :, h, :] = (
                    a1 * o_part[b, :, h, :] + a2 * o_xrecv[b, :, h, :]) / ln
            oy = pltpu.make_async_remote_copy(
                src_ref=o_comb.at[pl.ds(b, 1)],
                dst_ref=o_yrecv.at[pl.ds(b, 1)],
                send_sem=oy_sems.at[b, 0], recv_sem=oy_sems.at[b, 1],
                device_id=(my_x, 1 - my_y),
                device_id_type=pl.DeviceIdType.MESH)
            oy.start()
            y_rdmas.append(oy)
            out_ref[pl.ds(b0 + b, 1)] = o_comb[pl.ds(b, 1)]

        pb = B - BH - b0
        for b in range(BH):
            y_rdmas[b].wait_recv()
            out_ref[pl.ds(pb + b, 1)] = o_yrecv[pl.ds(b, 1)]

        for b in range(BH):
            x_rdmas[b][0].wait_send()
            x_rdmas[b][1].wait_send()
            y_rdmas[b].wait_send()

    return pl.pallas_call(
        body,
        out_shape=jax.ShapeDtypeStruct((B, SQ, H, D), jnp.float32),
        in_specs=[
            pl.BlockSpec(memory_space=pltpu.VMEM),
            pl.BlockSpec(memory_space=pl.ANY),
            pl.BlockSpec(memory_space=pl.ANY),
        ],
        out_specs=pl.BlockSpec(memory_space=pltpu.VMEM),
        scratch_shapes=[
            pltpu.VMEM((4, SKV, D), jnp.float32),
            pltpu.VMEM((4, SKV, D), jnp.float32),
            pltpu.VMEM((BH, SQ, H, D), jnp.float32),
            pltpu.VMEM((BH, SQ, H, D), jnp.float32),
            pltpu.VMEM((BH, SQ, H, D), jnp.float32),
            pltpu.VMEM((BH, SQ, H, D), jnp.float32),
            pltpu.VMEM((BH, SQ, 2 * H), jnp.float32),
            pltpu.VMEM((BH, SQ, 2 * H), jnp.float32),
            pltpu.SemaphoreType.DMA((4, 2, 2)),
            pltpu.SemaphoreType.DMA((BH, 2)),
            pltpu.SemaphoreType.DMA((BH, 2)),
            pltpu.SemaphoreType.DMA((BH, 2)),
        ],
        compiler_params=pltpu.CompilerParams(collective_id=0),
    )(Q, K, V)
